# baseline (speedup 1.0000x reference)
import functools

import jax
import jax.numpy as jnp
from jax import lax
from jax.experimental import pallas as pl
from jax.experimental.pallas import tpu as pltpu
from jax.experimental.pallas import tpu_sc as plsc

NUM_EXPERTS = 16
TOP_K = 2
HIDDEN = 1024
MOE_FF = 768
TOKENS = 2048

NUM_ASSIGN = TOKENS * TOP_K
BLK = 128
NB = NUM_ASSIGN // BLK + NUM_EXPERTS
NR = NB * BLK

_SC_CORES = 2
_SC_SUBCORES = 16
_NW = _SC_CORES * _SC_SUBCORES

_DISPATCH_ROWS = NR // _NW
_DISPATCH_CHUNK = 48
_COMBINE_CHUNK = 32


def _router_body(x_ref, gate_ref, wsel_ref, onehot_ref):
    logits = jnp.dot(x_ref[...], gate_ref[...], preferred_element_type=jnp.float32)
    probs = jax.nn.softmax(logits, axis=-1)
    lane = lax.broadcasted_iota(jnp.int32, probs.shape, 1)
    m1 = jnp.max(probs, axis=-1, keepdims=True)
    i1 = jnp.min(jnp.where(probs == m1, lane, NUM_EXPERTS), axis=-1, keepdims=True)
    masked = jnp.where(lane == i1, -jnp.inf, probs)
    m2 = jnp.max(masked, axis=-1, keepdims=True)
    i2 = jnp.min(jnp.where(masked == m2, lane, NUM_EXPERTS), axis=-1, keepdims=True)
    denom = m1 + m2
    sel = (lane == i1) | (lane == i2)
    w = jnp.where(lane == i1, m1, jnp.where(lane == i2, m2, 0.0)) / denom
    wsel_ref[...] = w
    onehot_ref[...] = sel.astype(jnp.float32)


def _router(x, gate_w):
    return pl.pallas_call(
        _router_body,
        out_shape=(
            jax.ShapeDtypeStruct((TOKENS, NUM_EXPERTS), jnp.float32),
            jax.ShapeDtypeStruct((TOKENS, NUM_EXPERTS), jnp.float32),
        ),
    )(x, gate_w)


def _route_metadata(wsel, onehot):
    mask = onehot > 0.5
    maski = mask.astype(jnp.int32)
    csum = jnp.cumsum(maski, axis=0)
    counts = csum[-1]
    blocks = (counts + BLK - 1) // BLK
    blockstart = jnp.concatenate([jnp.zeros((1,), jnp.int32),
                                  jnp.cumsum(blocks)[:-1]]).astype(jnp.int32)
    base = blockstart * BLK

    posmat = base[None, :] + csum - 1
    flatpos = jnp.where(mask, posmat, NR)
    tok = lax.broadcasted_iota(jnp.int32, mask.shape, 0)
    src = jnp.zeros((NR + 1,), jnp.int32).at[flatpos.reshape(-1)].set(
        tok.reshape(-1), mode="drop")
    src_token = src[:NR]

    marks = jnp.zeros((NB,), jnp.int32).at[blockstart].max(
        jnp.arange(NUM_EXPERTS, dtype=jnp.int32), mode="drop")
    block_expert = lax.cummax(marks)
    bidx = jnp.arange(NB, dtype=jnp.int32)
    block_valid = jnp.clip(
        counts[block_expert] - (bidx - blockstart[block_expert]) * BLK, 0, BLK)

    first = jnp.argmax(maski, axis=1)
    last = (NUM_EXPERTS - 1) - jnp.argmax(maski[:, ::-1], axis=1)
    tidx = jnp.arange(TOKENS)
    pos0 = posmat[tidx, first]
    pos1 = posmat[tidx, last]
    w0 = wsel[tidx, first]
    w1 = wsel[tidx, last]
    return (src_token, block_expert, block_valid,
            pos0.astype(jnp.int32), pos1.astype(jnp.int32), w0, w1)


def _sc_mesh():
    return plsc.VectorSubcoreMesh(core_axis_name="c", subcore_axis_name="s")


@functools.cache
def _make_sc_gather(n_out, chunk):
    rows_per_worker = n_out // _NW
    n_chunks = rows_per_worker // chunk

    @functools.partial(
        pl.kernel,
        mesh=_sc_mesh(),
        out_type=jax.ShapeDtypeStruct((n_out, HIDDEN), jnp.float32),
        scratch_types=[
            pltpu.VMEM((rows_per_worker,), jnp.int32),
            pltpu.VMEM((chunk, HIDDEN), jnp.float32),
            pltpu.VMEM((chunk, HIDDEN), jnp.float32),
            pltpu.SemaphoreType.DMA,
            pltpu.SemaphoreType.DMA,
        ],
    )
    def _sc_gather(table_hbm, src_hbm, out_hbm, idx_v, rows0_v, rows1_v,
                   sem0, sem1):
        wid = lax.axis_index("s") * _SC_CORES + lax.axis_index("c")
        row_base = wid * rows_per_worker
        pltpu.sync_copy(src_hbm.at[pl.ds(row_base, rows_per_worker)], idx_v)

        bufs = ((rows0_v, sem0), (rows1_v, sem1))

        def start(c):
            buf, sem = bufs[c % 2]
            return pltpu.async_copy(
                table_hbm.at[idx_v.at[pl.ds(c * chunk, chunk)]], buf, sem)

        def drain(c, handle):
            handle.wait()
            pltpu.sync_copy(
                bufs[c % 2][0], out_hbm.at[pl.ds(row_base + c * chunk, chunk)])

        handles = [start(0)]
        for c in range(1, n_chunks):
            handles.append(start(c))
            drain(c - 1, handles[c - 1])
        drain(n_chunks - 1, handles[n_chunks - 1])

    return _sc_gather


def _group_ffn_body(be_ref, valid_ref, x_ref, wg_ref, wu_ref, wd_ref, out_ref):
    b = pl.program_id(0)

    @pl.when(valid_ref[b] > 0)
    def _():
        x = x_ref[...]
        g = jnp.dot(x, wg_ref[0], preferred_element_type=jnp.float32)
        u = jnp.dot(x, wu_ref[0], preferred_element_type=jnp.float32)
        h = (g * jax.nn.sigmoid(g)) * u
        out_ref[...] = jnp.dot(h, wd_ref[0], preferred_element_type=jnp.float32)


def _group_ffn(x_pad, w_gate, w_up, w_down, block_expert, block_valid):
    grid_spec = pltpu.PrefetchScalarGridSpec(
        num_scalar_prefetch=2,
        grid=(NB,),
        in_specs=[
            pl.BlockSpec((BLK, HIDDEN), lambda b, be, vd: (b, 0)),
            pl.BlockSpec((1, HIDDEN, MOE_FF), lambda b, be, vd: (be[b], 0, 0)),
            pl.BlockSpec((1, HIDDEN, MOE_FF), lambda b, be, vd: (be[b], 0, 0)),
            pl.BlockSpec((1, MOE_FF, HIDDEN), lambda b, be, vd: (be[b], 0, 0)),
        ],
        out_specs=pl.BlockSpec((BLK, HIDDEN), lambda b, be, vd: (b, 0)),
    )
    return pl.pallas_call(
        _group_ffn_body,
        grid_spec=grid_spec,
        out_shape=jax.ShapeDtypeStruct((NR, HIDDEN), jnp.float32),
    )(block_expert, block_valid, x_pad, w_gate, w_up, w_down)


_BT_COMB = 256


def _combine_body(y0_ref, y1_ref, w0_ref, w1_ref, out_ref):
    out_ref[...] = (y0_ref[...] * w0_ref[..., :1]
                    + y1_ref[...] * w1_ref[..., :1])


def _combine(y2, w0m, w1m):
    nblk = TOKENS // _BT_COMB
    return pl.pallas_call(
        _combine_body,
        grid=(nblk,),
        in_specs=[
            pl.BlockSpec((_BT_COMB, HIDDEN), lambda t: (t, 0)),
            pl.BlockSpec((_BT_COMB, HIDDEN), lambda t, n=nblk: (t + n, 0)),
            pl.BlockSpec((_BT_COMB, 128), lambda t: (t, 0)),
            pl.BlockSpec((_BT_COMB, 128), lambda t: (t, 0)),
        ],
        out_specs=pl.BlockSpec((_BT_COMB, HIDDEN), lambda t: (t, 0)),
        out_shape=jax.ShapeDtypeStruct((TOKENS, HIDDEN), jnp.float32),
    )(y2, y2, w0m, w1m)


@jax.jit
def kernel(hidden_states, gate_w, w_gate, w_up, w_down):
    wsel, onehot = _router(hidden_states, gate_w)
    (src_token, block_expert, block_valid,
     pos0, pos1, w0, w1) = _route_metadata(wsel, onehot)
    x_pad = _make_sc_gather(NR, _DISPATCH_CHUNK)(hidden_states, src_token)
    y_pad = _group_ffn(x_pad, w_gate, w_up, w_down, block_expert, block_valid)
    poscat = jnp.concatenate([pos0, pos1])
    y2 = _make_sc_gather(2 * TOKENS, _COMBINE_CHUNK)(y_pad, poscat)
    w0mat = jnp.tile(w0[:, None], (1, 128))
    w1mat = jnp.tile(w1[:, None], (1, 128))
    return _combine(y2, w0mat, w1mat)


# speedup vs baseline: 1.3148x; 1.3148x over previous
import functools

import jax
import jax.numpy as jnp
from jax import lax
from jax.experimental import pallas as pl
from jax.experimental.pallas import tpu as pltpu
from jax.experimental.pallas import tpu_sc as plsc

NUM_EXPERTS = 16
TOP_K = 2
HIDDEN = 1024
MOE_FF = 768
TOKENS = 2048

NUM_ASSIGN = TOKENS * TOP_K
BLK = 128
NB = NUM_ASSIGN // BLK + NUM_EXPERTS
NR = NB * BLK

_SC_CORES = 2
_SC_SUBCORES = 16
_NW = _SC_CORES * _SC_SUBCORES

_DISPATCH_ROWS = NR // _NW
_DISPATCH_CHUNK = 48
_COMBINE_CHUNK = 32


def _router_body(x_ref, gate_ref, wsel_ref, onehot_ref):
    logits = jnp.dot(x_ref[...], gate_ref[...], preferred_element_type=jnp.float32)
    probs = jax.nn.softmax(logits, axis=-1)
    lane = lax.broadcasted_iota(jnp.int32, probs.shape, 1)
    m1 = jnp.max(probs, axis=-1, keepdims=True)
    i1 = jnp.min(jnp.where(probs == m1, lane, NUM_EXPERTS), axis=-1, keepdims=True)
    masked = jnp.where(lane == i1, -jnp.inf, probs)
    m2 = jnp.max(masked, axis=-1, keepdims=True)
    i2 = jnp.min(jnp.where(masked == m2, lane, NUM_EXPERTS), axis=-1, keepdims=True)
    denom = m1 + m2
    sel = (lane == i1) | (lane == i2)
    w = jnp.where(lane == i1, m1, jnp.where(lane == i2, m2, 0.0)) / denom
    wsel_ref[...] = w
    onehot_ref[...] = sel.astype(jnp.float32)


def _router(x, gate_w):
    return pl.pallas_call(
        _router_body,
        out_shape=(
            jax.ShapeDtypeStruct((TOKENS, NUM_EXPERTS), jnp.float32),
            jax.ShapeDtypeStruct((TOKENS, NUM_EXPERTS), jnp.float32),
        ),
    )(x, gate_w)


def _route_metadata(wsel, onehot):
    mask = onehot > 0.5
    maski = mask.astype(jnp.int32)
    csum = jnp.cumsum(maski, axis=0)
    counts = csum[-1]
    blocks = (counts + BLK - 1) // BLK
    blockstart = jnp.concatenate([jnp.zeros((1,), jnp.int32),
                                  jnp.cumsum(blocks)[:-1]]).astype(jnp.int32)
    base = blockstart * BLK

    posmat = base[None, :] + csum - 1
    flatpos = jnp.where(mask, posmat, NR)
    tok = lax.broadcasted_iota(jnp.int32, mask.shape, 0)
    src = (jnp.arange(NR + 1, dtype=jnp.int32) % TOKENS).at[
        flatpos.reshape(-1)].set(tok.reshape(-1), mode="drop")
    src_token = src[:NR]

    marks = jnp.zeros((NB,), jnp.int32).at[blockstart].max(
        jnp.arange(NUM_EXPERTS, dtype=jnp.int32), mode="drop")
    block_expert = lax.cummax(marks)
    bidx = jnp.arange(NB, dtype=jnp.int32)
    block_valid = jnp.clip(
        counts[block_expert] - (bidx - blockstart[block_expert]) * BLK, 0, BLK)

    first = jnp.argmax(maski, axis=1)
    last = (NUM_EXPERTS - 1) - jnp.argmax(maski[:, ::-1], axis=1)
    tidx = jnp.arange(TOKENS)
    pos0 = posmat[tidx, first]
    pos1 = posmat[tidx, last]
    w0 = wsel[tidx, first]
    w1 = wsel[tidx, last]
    return (src_token, block_expert, block_valid,
            pos0.astype(jnp.int32), pos1.astype(jnp.int32), w0, w1)


def _sc_mesh():
    return plsc.VectorSubcoreMesh(core_axis_name="c", subcore_axis_name="s")


@functools.cache
def _make_sc_gather(n_out, chunk):
    rows_per_worker = n_out // _NW
    n_chunks = rows_per_worker // chunk

    @functools.partial(
        pl.kernel,
        mesh=_sc_mesh(),
        out_type=jax.ShapeDtypeStruct((n_out, HIDDEN), jnp.float32),
        scratch_types=[
            pltpu.VMEM((rows_per_worker,), jnp.int32),
            pltpu.VMEM((chunk, HIDDEN), jnp.float32),
            pltpu.VMEM((chunk, HIDDEN), jnp.float32),
            pltpu.SemaphoreType.DMA,
            pltpu.SemaphoreType.DMA,
        ],
    )
    def _sc_gather(table_hbm, src_hbm, out_hbm, idx_v, rows0_v, rows1_v,
                   sem0, sem1):
        wid = lax.axis_index("s") * _SC_CORES + lax.axis_index("c")
        row_base = wid * rows_per_worker
        pltpu.sync_copy(src_hbm.at[pl.ds(row_base, rows_per_worker)], idx_v)

        bufs = ((rows0_v, sem0), (rows1_v, sem1))

        def start(c):
            buf, sem = bufs[c % 2]
            return pltpu.async_copy(
                table_hbm.at[idx_v.at[pl.ds(c * chunk, chunk)]], buf, sem)

        def drain(c, handle):
            handle.wait()
            pltpu.sync_copy(
                bufs[c % 2][0], out_hbm.at[pl.ds(row_base + c * chunk, chunk)])

        handles = [start(0)]
        for c in range(1, n_chunks):
            handles.append(start(c))
            drain(c - 1, handles[c - 1])
        drain(n_chunks - 1, handles[n_chunks - 1])

    return _sc_gather


def _group_ffn_body(be_ref, valid_ref, x_ref, wg_ref, wu_ref, wd_ref, out_ref):
    b = pl.program_id(0)

    @pl.when(valid_ref[b] > 0)
    def _():
        x = x_ref[...]
        g = jnp.dot(x, wg_ref[0], preferred_element_type=jnp.float32)
        u = jnp.dot(x, wu_ref[0], preferred_element_type=jnp.float32)
        h = (g * jax.nn.sigmoid(g)) * u
        out_ref[...] = jnp.dot(h, wd_ref[0], preferred_element_type=jnp.float32)


def _group_ffn(x_pad, w_gate, w_up, w_down, block_expert, block_valid):
    grid_spec = pltpu.PrefetchScalarGridSpec(
        num_scalar_prefetch=2,
        grid=(NB,),
        in_specs=[
            pl.BlockSpec((BLK, HIDDEN), lambda b, be, vd: (b, 0)),
            pl.BlockSpec((1, HIDDEN, MOE_FF), lambda b, be, vd: (be[b], 0, 0)),
            pl.BlockSpec((1, HIDDEN, MOE_FF), lambda b, be, vd: (be[b], 0, 0)),
            pl.BlockSpec((1, MOE_FF, HIDDEN), lambda b, be, vd: (be[b], 0, 0)),
        ],
        out_specs=pl.BlockSpec((BLK, HIDDEN), lambda b, be, vd: (b, 0)),
    )
    return pl.pallas_call(
        _group_ffn_body,
        grid_spec=grid_spec,
        out_shape=jax.ShapeDtypeStruct((NR, HIDDEN), jnp.float32),
    )(block_expert, block_valid, x_pad, w_gate, w_up, w_down)


_BT_COMB = 256


def _combine_body(y0_ref, y1_ref, w0_ref, w1_ref, out_ref):
    out_ref[...] = (y0_ref[...] * w0_ref[..., :1]
                    + y1_ref[...] * w1_ref[..., :1])


def _combine(y2, w0m, w1m):
    nblk = TOKENS // _BT_COMB
    return pl.pallas_call(
        _combine_body,
        grid=(nblk,),
        in_specs=[
            pl.BlockSpec((_BT_COMB, HIDDEN), lambda t: (t, 0)),
            pl.BlockSpec((_BT_COMB, HIDDEN), lambda t, n=nblk: (t + n, 0)),
            pl.BlockSpec((_BT_COMB, 128), lambda t: (t, 0)),
            pl.BlockSpec((_BT_COMB, 128), lambda t: (t, 0)),
        ],
        out_specs=pl.BlockSpec((_BT_COMB, HIDDEN), lambda t: (t, 0)),
        out_shape=jax.ShapeDtypeStruct((TOKENS, HIDDEN), jnp.float32),
    )(y2, y2, w0m, w1m)


@jax.jit
def kernel(hidden_states, gate_w, w_gate, w_up, w_down):
    wsel, onehot = _router(hidden_states, gate_w)
    (src_token, block_expert, block_valid,
     pos0, pos1, w0, w1) = _route_metadata(wsel, onehot)
    x_pad = _make_sc_gather(NR, _DISPATCH_CHUNK)(hidden_states, src_token)
    y_pad = _group_ffn(x_pad, w_gate, w_up, w_down, block_expert, block_valid)
    poscat = jnp.concatenate([pos0, pos1])
    y2 = _make_sc_gather(2 * TOKENS, _COMBINE_CHUNK)(y_pad, poscat)
    w0mat = jnp.tile(w0[:, None], (1, 128))
    w1mat = jnp.tile(w1[:, None], (1, 128))
    return _combine(y2, w0mat, w1mat)


# speedup vs baseline: 1.4888x; 1.1324x over previous
import functools

import jax
import jax.numpy as jnp
from jax import lax
from jax.experimental import pallas as pl
from jax.experimental.pallas import tpu as pltpu
from jax.experimental.pallas import tpu_sc as plsc

NUM_EXPERTS = 16
TOP_K = 2
HIDDEN = 1024
MOE_FF = 768
TOKENS = 2048

NUM_ASSIGN = TOKENS * TOP_K
BLK = 128
NB = NUM_ASSIGN // BLK + NUM_EXPERTS
NR = NB * BLK

_SC_CORES = 2
_SC_SUBCORES = 16
_NW = _SC_CORES * _SC_SUBCORES

_DISPATCH_ROWS = NR // _NW
_DISPATCH_CHUNK = 48
_COMBINE_CHUNK = 32


_CS_CHUNK = 256


def _router_meta_body(x_ref, gate_ref, flatpos_ref, pospair_ref,
                      w0m_ref, w1m_ref, counts_ref):
    logits = jnp.dot(x_ref[...], gate_ref[...], preferred_element_type=jnp.float32)
    probs = jax.nn.softmax(logits, axis=-1)
    lane = lax.broadcasted_iota(jnp.int32, probs.shape, 1)
    m1 = jnp.max(probs, axis=-1, keepdims=True)
    i1 = jnp.min(jnp.where(probs == m1, lane, NUM_EXPERTS), axis=-1, keepdims=True)
    masked = jnp.where(lane == i1, -jnp.inf, probs)
    m2 = jnp.max(masked, axis=-1, keepdims=True)
    i2 = jnp.min(jnp.where(masked == m2, lane, NUM_EXPERTS), axis=-1, keepdims=True)
    denom = m1 + m2
    sel = (lane == i1) | (lane == i2)
    w = jnp.where(lane == i1, m1, jnp.where(lane == i2, m2, 0.0)) / denom

    m = sel.astype(jnp.float32)
    tri_r = lax.broadcasted_iota(jnp.int32, (_CS_CHUNK, _CS_CHUNK), 0)
    tri_c = lax.broadcasted_iota(jnp.int32, (_CS_CHUNK, _CS_CHUNK), 1)
    tri = (tri_r >= tri_c).astype(jnp.float32)
    carry = jnp.zeros((1, NUM_EXPERTS), jnp.float32)
    chunks = []
    for c in range(TOKENS // _CS_CHUNK):
        mc = m[c * _CS_CHUNK:(c + 1) * _CS_CHUNK]
        cs = jnp.dot(tri, mc, preferred_element_type=jnp.float32) + carry
        chunks.append(cs)
        carry = cs[_CS_CHUNK - 1:_CS_CHUNK, :]
    csum = jnp.concatenate(chunks, axis=0)
    counts = carry

    counts_i = counts.astype(jnp.int32)
    blocks_f = ((counts_i + (BLK - 1)) >> 7).astype(jnp.float32)
    blocks8 = jnp.broadcast_to(blocks_f, (8, NUM_EXPERTS))
    ut_r = lax.broadcasted_iota(jnp.int32, (NUM_EXPERTS, NUM_EXPERTS), 0)
    ut_c = lax.broadcasted_iota(jnp.int32, (NUM_EXPERTS, NUM_EXPERTS), 1)
    ut = (ut_r < ut_c).astype(jnp.float32)
    bstart8 = jnp.dot(blocks8, ut, preferred_element_type=jnp.float32)
    base = bstart8[0:1, :] * BLK

    posmat = base + csum - 1.0
    posmat_i = posmat.astype(jnp.int32)
    flatpos_ref[...] = jnp.where(sel, posmat_i, NR)

    efirst = jnp.minimum(i1, i2)
    elast = jnp.maximum(i1, i2)
    pos0 = jnp.sum(jnp.where(lane == efirst, posmat, 0.0), axis=1, keepdims=True)
    pos1 = jnp.sum(jnp.where(lane == elast, posmat, 0.0), axis=1, keepdims=True)
    w0 = jnp.sum(jnp.where(lane == efirst, w, 0.0), axis=1, keepdims=True)
    w1 = jnp.sum(jnp.where(lane == elast, w, 0.0), axis=1, keepdims=True)
    lane8 = lax.broadcasted_iota(jnp.int32, (TOKENS, 8), 1)
    pospair_ref[...] = jnp.where(
        lane8 == 0, pos0.astype(jnp.int32),
        jnp.where(lane8 == 1, pos1.astype(jnp.int32), 0))
    w0m_ref[...] = jnp.broadcast_to(w0, (TOKENS, NUM_EXPERTS))
    w1m_ref[...] = jnp.broadcast_to(w1, (TOKENS, NUM_EXPERTS))
    counts_ref[...] = jnp.broadcast_to(counts_i, (8, NUM_EXPERTS))


def _router_meta(x, gate_w):
    return pl.pallas_call(
        _router_meta_body,
        out_shape=(
            jax.ShapeDtypeStruct((TOKENS, NUM_EXPERTS), jnp.int32),
            jax.ShapeDtypeStruct((TOKENS, 8), jnp.int32),
            jax.ShapeDtypeStruct((TOKENS, NUM_EXPERTS), jnp.float32),
            jax.ShapeDtypeStruct((TOKENS, NUM_EXPERTS), jnp.float32),
            jax.ShapeDtypeStruct((8, NUM_EXPERTS), jnp.int32),
        ),
    )(x, gate_w)


def _route_glue(flatpos, pospair, counts8):
    counts = counts8[0]
    blocks = (counts + BLK - 1) // BLK
    blockstart = jnp.concatenate([jnp.zeros((1,), jnp.int32),
                                  jnp.cumsum(blocks)[:-1]]).astype(jnp.int32)
    bidx = jnp.arange(NB, dtype=jnp.int32)
    block_expert = jnp.sum(
        (blockstart[None, :] <= bidx[:, None]).astype(jnp.int32), axis=1) - 1
    block_expert = jnp.clip(block_expert, 0, NUM_EXPERTS - 1)
    block_valid = jnp.clip(
        counts[block_expert] - (bidx - blockstart[block_expert]) * BLK, 0, BLK)

    tok = lax.broadcasted_iota(jnp.int32, (TOKENS, NUM_EXPERTS), 0)
    src = (jnp.arange(NR + 1, dtype=jnp.int32) % TOKENS).at[
        flatpos.reshape(-1)].set(tok.reshape(-1), mode="drop")
    src_token = src[:NR]
    pos0 = pospair[:, 0]
    pos1 = pospair[:, 1]
    return src_token, block_expert, block_valid, pos0, pos1


def _sc_mesh():
    return plsc.VectorSubcoreMesh(core_axis_name="c", subcore_axis_name="s")


@functools.cache
def _make_sc_gather(n_out, chunk):
    rows_per_worker = n_out // _NW
    n_chunks = rows_per_worker // chunk

    @functools.partial(
        pl.kernel,
        mesh=_sc_mesh(),
        out_type=jax.ShapeDtypeStruct((n_out, HIDDEN), jnp.float32),
        scratch_types=[
            pltpu.VMEM((rows_per_worker,), jnp.int32),
            pltpu.VMEM((chunk, HIDDEN), jnp.float32),
            pltpu.VMEM((chunk, HIDDEN), jnp.float32),
            pltpu.SemaphoreType.DMA,
            pltpu.SemaphoreType.DMA,
        ],
    )
    def _sc_gather(table_hbm, src_hbm, out_hbm, idx_v, rows0_v, rows1_v,
                   sem0, sem1):
        wid = lax.axis_index("s") * _SC_CORES + lax.axis_index("c")
        row_base = wid * rows_per_worker
        pltpu.sync_copy(src_hbm.at[pl.ds(row_base, rows_per_worker)], idx_v)

        bufs = ((rows0_v, sem0), (rows1_v, sem1))

        def start(c):
            buf, sem = bufs[c % 2]
            return pltpu.async_copy(
                table_hbm.at[idx_v.at[pl.ds(c * chunk, chunk)]], buf, sem)

        def drain(c, handle):
            handle.wait()
            pltpu.sync_copy(
                bufs[c % 2][0], out_hbm.at[pl.ds(row_base + c * chunk, chunk)])

        handles = [start(0)]
        for c in range(1, n_chunks):
            handles.append(start(c))
            drain(c - 1, handles[c - 1])
        drain(n_chunks - 1, handles[n_chunks - 1])

    return _sc_gather


def _group_ffn_body(be_ref, valid_ref, x_ref, wg_ref, wu_ref, wd_ref, out_ref):
    b = pl.program_id(0)

    @pl.when(valid_ref[b] > 0)
    def _():
        x = x_ref[...]
        g = jnp.dot(x, wg_ref[0], preferred_element_type=jnp.float32)
        u = jnp.dot(x, wu_ref[0], preferred_element_type=jnp.float32)
        h = (g * jax.nn.sigmoid(g)) * u
        out_ref[...] = jnp.dot(h, wd_ref[0], preferred_element_type=jnp.float32)


def _group_ffn(x_pad, w_gate, w_up, w_down, block_expert, block_valid):
    grid_spec = pltpu.PrefetchScalarGridSpec(
        num_scalar_prefetch=2,
        grid=(NB,),
        in_specs=[
            pl.BlockSpec((BLK, HIDDEN), lambda b, be, vd: (b, 0)),
            pl.BlockSpec((1, HIDDEN, MOE_FF), lambda b, be, vd: (be[b], 0, 0)),
            pl.BlockSpec((1, HIDDEN, MOE_FF), lambda b, be, vd: (be[b], 0, 0)),
            pl.BlockSpec((1, MOE_FF, HIDDEN), lambda b, be, vd: (be[b], 0, 0)),
        ],
        out_specs=pl.BlockSpec((BLK, HIDDEN), lambda b, be, vd: (b, 0)),
    )
    return pl.pallas_call(
        _group_ffn_body,
        grid_spec=grid_spec,
        out_shape=jax.ShapeDtypeStruct((NR, HIDDEN), jnp.float32),
    )(block_expert, block_valid, x_pad, w_gate, w_up, w_down)


_COMBINE_TOKENS = TOKENS // _NW


@functools.cache
def _make_sc_combine():
    @functools.partial(
        pl.kernel,
        mesh=_sc_mesh(),
        out_type=jax.ShapeDtypeStruct((TOKENS, HIDDEN), jnp.float32),
        scratch_types=[
            pltpu.VMEM((_COMBINE_TOKENS,), jnp.int32),
            pltpu.VMEM((_COMBINE_TOKENS,), jnp.int32),
            pltpu.VMEM((_COMBINE_TOKENS, NUM_EXPERTS), jnp.float32),
            pltpu.VMEM((_COMBINE_TOKENS, NUM_EXPERTS), jnp.float32),
            pltpu.VMEM((_COMBINE_CHUNK, HIDDEN), jnp.float32),
            pltpu.VMEM((_COMBINE_CHUNK, HIDDEN), jnp.float32),
            pltpu.VMEM((_COMBINE_CHUNK, HIDDEN), jnp.float32),
            pltpu.SemaphoreType.DMA,
            pltpu.SemaphoreType.DMA,
        ],
    )
    def _sc_combine(y_hbm, pos0_hbm, pos1_hbm, w0_hbm, w1_hbm, out_hbm,
                    pos0_v, pos1_v, w0_v, w1_v, y0_v, y1_v, o_v, sem0, sem1):
        wid = lax.axis_index("s") * _SC_CORES + lax.axis_index("c")
        tok_base = wid * _COMBINE_TOKENS
        pltpu.sync_copy(pos0_hbm.at[pl.ds(tok_base, _COMBINE_TOKENS)], pos0_v)
        pltpu.sync_copy(pos1_hbm.at[pl.ds(tok_base, _COMBINE_TOKENS)], pos1_v)
        pltpu.sync_copy(w0_hbm.at[pl.ds(tok_base, _COMBINE_TOKENS)], w0_v)
        pltpu.sync_copy(w1_hbm.at[pl.ds(tok_base, _COMBINE_TOKENS)], w1_v)

        def chunk(c, _):
            off = c * _COMBINE_CHUNK
            cp0 = pltpu.async_copy(
                y_hbm.at[pos0_v.at[pl.ds(off, _COMBINE_CHUNK)]], y0_v, sem0)
            cp1 = pltpu.async_copy(
                y_hbm.at[pos1_v.at[pl.ds(off, _COMBINE_CHUNK)]], y1_v, sem1)
            cp0.wait()
            cp1.wait()

            def token(j, _):
                wa = w0_v[off + j, :]
                wb = w1_v[off + j, :]
                for i in range(HIDDEN // 16):
                    sl = pl.ds(i * 16, 16)
                    o_v[j, sl] = wa * y0_v[j, sl] + wb * y1_v[j, sl]
                return 0

            lax.fori_loop(0, _COMBINE_CHUNK, token, 0)
            pltpu.sync_copy(
                o_v, out_hbm.at[pl.ds(tok_base + off, _COMBINE_CHUNK)])
            return 0

        lax.fori_loop(0, _COMBINE_TOKENS // _COMBINE_CHUNK, chunk, 0)

    return _sc_combine


@jax.jit
def kernel(hidden_states, gate_w, w_gate, w_up, w_down):
    flatpos, pospair, w0m, w1m, counts8 = _router_meta(hidden_states, gate_w)
    (src_token, block_expert, block_valid,
     pos0, pos1) = _route_glue(flatpos, pospair, counts8)
    x_pad = _make_sc_gather(NR, _DISPATCH_CHUNK)(hidden_states, src_token)
    y_pad = _group_ffn(x_pad, w_gate, w_up, w_down, block_expert, block_valid)
    return _make_sc_combine()(y_pad, pos0, pos1, w0m, w1m)


# speedup vs baseline: 2.6321x; 1.7679x over previous
import functools

import jax
import jax.numpy as jnp
from jax import lax
from jax.experimental import pallas as pl
from jax.experimental.pallas import tpu as pltpu
from jax.experimental.pallas import tpu_sc as plsc

NUM_EXPERTS = 16
TOP_K = 2
HIDDEN = 1024
MOE_FF = 768
TOKENS = 2048

NUM_ASSIGN = TOKENS * TOP_K
BLK = 128
NB = NUM_ASSIGN // BLK + NUM_EXPERTS
NR = NB * BLK

_SC_CORES = 2
_SC_SUBCORES = 16
_NW = _SC_CORES * _SC_SUBCORES

_DISPATCH_ROWS = NR // _NW
_DISPATCH_CHUNK = 48
_COMBINE_CHUNK = 32


_CS_CHUNK = 256


def _router_meta_body(x_ref, gate_ref, pospair_ref,
                      w0m_ref, w1m_ref, counts_ref):
    logits = jnp.dot(x_ref[...], gate_ref[...], preferred_element_type=jnp.float32)
    probs = jax.nn.softmax(logits, axis=-1)
    lane = lax.broadcasted_iota(jnp.int32, probs.shape, 1)
    m1 = jnp.max(probs, axis=-1, keepdims=True)
    i1 = jnp.min(jnp.where(probs == m1, lane, NUM_EXPERTS), axis=-1, keepdims=True)
    masked = jnp.where(lane == i1, -jnp.inf, probs)
    m2 = jnp.max(masked, axis=-1, keepdims=True)
    i2 = jnp.min(jnp.where(masked == m2, lane, NUM_EXPERTS), axis=-1, keepdims=True)
    denom = m1 + m2
    sel = (lane == i1) | (lane == i2)
    w = jnp.where(lane == i1, m1, jnp.where(lane == i2, m2, 0.0)) / denom

    m = sel.astype(jnp.float32)
    tri_r = lax.broadcasted_iota(jnp.int32, (_CS_CHUNK, _CS_CHUNK), 0)
    tri_c = lax.broadcasted_iota(jnp.int32, (_CS_CHUNK, _CS_CHUNK), 1)
    tri = (tri_r >= tri_c).astype(jnp.float32)
    carry = jnp.zeros((1, NUM_EXPERTS), jnp.float32)
    chunks = []
    for c in range(TOKENS // _CS_CHUNK):
        mc = m[c * _CS_CHUNK:(c + 1) * _CS_CHUNK]
        cs = jnp.dot(tri, mc, preferred_element_type=jnp.float32) + carry
        chunks.append(cs)
        carry = cs[_CS_CHUNK - 1:_CS_CHUNK, :]
    csum = jnp.concatenate(chunks, axis=0)
    counts = carry

    counts_i = counts.astype(jnp.int32)
    blocks_f = ((counts_i + (BLK - 1)) >> 7).astype(jnp.float32)
    blocks8 = jnp.broadcast_to(blocks_f, (8, NUM_EXPERTS))
    ut_r = lax.broadcasted_iota(jnp.int32, (NUM_EXPERTS, NUM_EXPERTS), 0)
    ut_c = lax.broadcasted_iota(jnp.int32, (NUM_EXPERTS, NUM_EXPERTS), 1)
    ut = (ut_r < ut_c).astype(jnp.float32)
    bstart8 = jnp.dot(blocks8, ut, preferred_element_type=jnp.float32)
    base = bstart8[0:1, :] * BLK

    posmat = base + csum - 1.0
    efirst = jnp.minimum(i1, i2)
    elast = jnp.maximum(i1, i2)
    pos0 = jnp.sum(jnp.where(lane == efirst, posmat, 0.0), axis=1, keepdims=True)
    pos1 = jnp.sum(jnp.where(lane == elast, posmat, 0.0), axis=1, keepdims=True)
    w0 = jnp.sum(jnp.where(lane == efirst, w, 0.0), axis=1, keepdims=True)
    w1 = jnp.sum(jnp.where(lane == elast, w, 0.0), axis=1, keepdims=True)
    lane8 = lax.broadcasted_iota(jnp.int32, (TOKENS, 8), 1)
    pospair_ref[...] = jnp.where(
        lane8 == 0, pos0.astype(jnp.int32),
        jnp.where(lane8 == 1, pos1.astype(jnp.int32), 0))
    w0m_ref[...] = jnp.broadcast_to(w0, (TOKENS, NUM_EXPERTS))
    w1m_ref[...] = jnp.broadcast_to(w1, (TOKENS, NUM_EXPERTS))
    counts_ref[...] = jnp.broadcast_to(counts_i, (8, NUM_EXPERTS))


def _router_meta(x, gate_w):
    return pl.pallas_call(
        _router_meta_body,
        out_shape=(
            jax.ShapeDtypeStruct((TOKENS, 8), jnp.int32),
            jax.ShapeDtypeStruct((TOKENS, NUM_EXPERTS), jnp.float32),
            jax.ShapeDtypeStruct((TOKENS, NUM_EXPERTS), jnp.float32),
            jax.ShapeDtypeStruct((8, NUM_EXPERTS), jnp.int32),
        ),
    )(x, gate_w)


def _route_glue(pospair, counts8):
    counts = counts8[0]
    blocks = (counts + BLK - 1) // BLK
    blockstart = jnp.concatenate([jnp.zeros((1,), jnp.int32),
                                  jnp.cumsum(blocks)[:-1]]).astype(jnp.int32)
    bidx = jnp.arange(NB, dtype=jnp.int32)
    block_expert = jnp.sum(
        (blockstart[None, :] <= bidx[:, None]).astype(jnp.int32), axis=1) - 1
    block_expert = jnp.clip(block_expert, 0, NUM_EXPERTS - 1)
    block_valid = jnp.clip(
        counts[block_expert] - (bidx - blockstart[block_expert]) * BLK, 0, BLK)

    pos0 = pospair[:, 0]
    pos1 = pospair[:, 1]
    return block_expert, block_valid, pos0, pos1


def _sc_mesh():
    return plsc.VectorSubcoreMesh(core_axis_name="c", subcore_axis_name="s")


_DISP_TOK = TOKENS // _NW


@functools.cache
def _make_sc_dispatch():
    @functools.partial(
        pl.kernel,
        mesh=_sc_mesh(),
        out_type=jax.ShapeDtypeStruct((NR, HIDDEN), jnp.float32),
        scratch_types=[
            pltpu.VMEM((_DISP_TOK,), jnp.int32),
            pltpu.VMEM((_DISP_TOK,), jnp.int32),
            pltpu.VMEM((_DISP_TOK, HIDDEN), jnp.float32),
            pltpu.SemaphoreType.DMA,
            pltpu.SemaphoreType.DMA,
        ],
    )
    def _sc_dispatch(hid_hbm, pos0_hbm, pos1_hbm, out_hbm,
                     idx0_v, idx1_v, rows_v, sem0, sem1):
        wid = lax.axis_index("s") * _SC_CORES + lax.axis_index("c")
        tok_base = wid * _DISP_TOK
        pltpu.sync_copy(pos0_hbm.at[wid], idx0_v)
        pltpu.sync_copy(pos1_hbm.at[wid], idx1_v)
        pltpu.sync_copy(hid_hbm.at[pl.ds(tok_base, _DISP_TOK)], rows_v)
        cp0 = pltpu.async_copy(rows_v, out_hbm.at[idx0_v], sem0)
        cp1 = pltpu.async_copy(rows_v, out_hbm.at[idx1_v], sem1)
        cp0.wait()
        cp1.wait()

    return _sc_dispatch


def _group_ffn_body(be_ref, valid_ref, x_ref, wg_ref, wu_ref, wd_ref, out_ref):
    b = pl.program_id(0)

    @pl.when(valid_ref[b] > 0)
    def _():
        x = x_ref[...]
        g = jnp.dot(x, wg_ref[0], preferred_element_type=jnp.float32)
        u = jnp.dot(x, wu_ref[0], preferred_element_type=jnp.float32)
        h = (g * jax.nn.sigmoid(g)) * u
        out_ref[...] = jnp.dot(h, wd_ref[0], preferred_element_type=jnp.float32)


def _group_ffn(x_pad, w_gate, w_up, w_down, block_expert, block_valid):
    grid_spec = pltpu.PrefetchScalarGridSpec(
        num_scalar_prefetch=2,
        grid=(NB,),
        in_specs=[
            pl.BlockSpec((BLK, HIDDEN), lambda b, be, vd: (b, 0)),
            pl.BlockSpec((1, HIDDEN, MOE_FF), lambda b, be, vd: (be[b], 0, 0)),
            pl.BlockSpec((1, HIDDEN, MOE_FF), lambda b, be, vd: (be[b], 0, 0)),
            pl.BlockSpec((1, MOE_FF, HIDDEN), lambda b, be, vd: (be[b], 0, 0)),
        ],
        out_specs=pl.BlockSpec((BLK, HIDDEN), lambda b, be, vd: (b, 0)),
    )
    return pl.pallas_call(
        _group_ffn_body,
        grid_spec=grid_spec,
        out_shape=jax.ShapeDtypeStruct((NR, HIDDEN), jnp.float32),
    )(block_expert, block_valid, x_pad, w_gate, w_up, w_down)


_COMBINE_TOKENS = TOKENS // _NW


@functools.cache
def _make_sc_combine():
    @functools.partial(
        pl.kernel,
        mesh=_sc_mesh(),
        out_type=jax.ShapeDtypeStruct((TOKENS, HIDDEN), jnp.float32),
        scratch_types=[
            pltpu.VMEM((_COMBINE_TOKENS,), jnp.int32),
            pltpu.VMEM((_COMBINE_TOKENS,), jnp.int32),
            pltpu.VMEM((_COMBINE_TOKENS, NUM_EXPERTS), jnp.float32),
            pltpu.VMEM((_COMBINE_TOKENS, NUM_EXPERTS), jnp.float32),
            pltpu.VMEM((_COMBINE_CHUNK, HIDDEN), jnp.float32),
            pltpu.VMEM((_COMBINE_CHUNK, HIDDEN), jnp.float32),
            pltpu.VMEM((_COMBINE_CHUNK, HIDDEN), jnp.float32),
            pltpu.SemaphoreType.DMA,
            pltpu.SemaphoreType.DMA,
        ],
    )
    def _sc_combine(y_hbm, pos0_hbm, pos1_hbm, w0_hbm, w1_hbm, out_hbm,
                    pos0_v, pos1_v, w0_v, w1_v, y0_v, y1_v, o_v, sem0, sem1):
        wid = lax.axis_index("s") * _SC_CORES + lax.axis_index("c")
        tok_base = wid * _COMBINE_TOKENS
        pltpu.sync_copy(pos0_hbm.at[pl.ds(tok_base, _COMBINE_TOKENS)], pos0_v)
        pltpu.sync_copy(pos1_hbm.at[pl.ds(tok_base, _COMBINE_TOKENS)], pos1_v)
        pltpu.sync_copy(w0_hbm.at[pl.ds(tok_base, _COMBINE_TOKENS)], w0_v)
        pltpu.sync_copy(w1_hbm.at[pl.ds(tok_base, _COMBINE_TOKENS)], w1_v)

        def chunk(c, _):
            off = c * _COMBINE_CHUNK
            cp0 = pltpu.async_copy(
                y_hbm.at[pos0_v.at[pl.ds(off, _COMBINE_CHUNK)]], y0_v, sem0)
            cp1 = pltpu.async_copy(
                y_hbm.at[pos1_v.at[pl.ds(off, _COMBINE_CHUNK)]], y1_v, sem1)
            cp0.wait()
            cp1.wait()

            def token(j, _):
                wa = w0_v[off + j, :]
                wb = w1_v[off + j, :]
                for i in range(HIDDEN // 16):
                    sl = pl.ds(i * 16, 16)
                    o_v[j, sl] = wa * y0_v[j, sl] + wb * y1_v[j, sl]
                return 0

            lax.fori_loop(0, _COMBINE_CHUNK, token, 0)
            pltpu.sync_copy(
                o_v, out_hbm.at[pl.ds(tok_base + off, _COMBINE_CHUNK)])
            return 0

        lax.fori_loop(0, _COMBINE_TOKENS // _COMBINE_CHUNK, chunk, 0)

    return _sc_combine


@jax.jit
def kernel(hidden_states, gate_w, w_gate, w_up, w_down):
    pospair, w0m, w1m, counts8 = _router_meta(hidden_states, gate_w)
    block_expert, block_valid, pos0, pos1 = _route_glue(pospair, counts8)
    x_pad = _make_sc_dispatch()(
        hidden_states, pos0.reshape(_NW, _DISP_TOK), pos1.reshape(_NW, _DISP_TOK))
    y_pad = _group_ffn(x_pad, w_gate, w_up, w_down, block_expert, block_valid)
    return _make_sc_combine()(y_pad, pos0, pos1, w0m, w1m)


# speedup vs baseline: 2.7149x; 1.0315x over previous
import functools

import jax
import jax.numpy as jnp
from jax import lax
from jax.experimental import pallas as pl
from jax.experimental.pallas import tpu as pltpu
from jax.experimental.pallas import tpu_sc as plsc

NUM_EXPERTS = 16
TOP_K = 2
HIDDEN = 1024
MOE_FF = 768
TOKENS = 2048

NUM_ASSIGN = TOKENS * TOP_K
BLK = 128
NB = NUM_ASSIGN // BLK + NUM_EXPERTS
NR = NB * BLK

_SC_CORES = 2
_SC_SUBCORES = 16
_NW = _SC_CORES * _SC_SUBCORES

_DISPATCH_ROWS = NR // _NW
_DISPATCH_CHUNK = 48
_COMBINE_CHUNK = 32


_CS_CHUNK = 256


def _router_meta_body(x_ref, gate_ref, pospair_ref,
                      w0m_ref, w1m_ref, counts_ref):
    logits = jnp.dot(x_ref[...], gate_ref[...], preferred_element_type=jnp.float32)
    probs = jax.nn.softmax(logits, axis=-1)
    lane = lax.broadcasted_iota(jnp.int32, probs.shape, 1)
    m1 = jnp.max(probs, axis=-1, keepdims=True)
    i1 = jnp.min(jnp.where(probs == m1, lane, NUM_EXPERTS), axis=-1, keepdims=True)
    masked = jnp.where(lane == i1, -jnp.inf, probs)
    m2 = jnp.max(masked, axis=-1, keepdims=True)
    i2 = jnp.min(jnp.where(masked == m2, lane, NUM_EXPERTS), axis=-1, keepdims=True)
    denom = m1 + m2
    sel = (lane == i1) | (lane == i2)
    w = jnp.where(lane == i1, m1, jnp.where(lane == i2, m2, 0.0)) / denom

    m = sel.astype(jnp.float32)
    tri_r = lax.broadcasted_iota(jnp.int32, (_CS_CHUNK, _CS_CHUNK), 0)
    tri_c = lax.broadcasted_iota(jnp.int32, (_CS_CHUNK, _CS_CHUNK), 1)
    tri = (tri_r >= tri_c).astype(jnp.float32)
    carry = jnp.zeros((1, NUM_EXPERTS), jnp.float32)
    chunks = []
    for c in range(TOKENS // _CS_CHUNK):
        mc = m[c * _CS_CHUNK:(c + 1) * _CS_CHUNK]
        cs = jnp.dot(tri, mc, preferred_element_type=jnp.float32) + carry
        chunks.append(cs)
        carry = cs[_CS_CHUNK - 1:_CS_CHUNK, :]
    csum = jnp.concatenate(chunks, axis=0)
    counts = carry

    counts_i = counts.astype(jnp.int32)
    blocks_f = ((counts_i + (BLK - 1)) >> 7).astype(jnp.float32)
    blocks8 = jnp.broadcast_to(blocks_f, (8, NUM_EXPERTS))
    ut_r = lax.broadcasted_iota(jnp.int32, (NUM_EXPERTS, NUM_EXPERTS), 0)
    ut_c = lax.broadcasted_iota(jnp.int32, (NUM_EXPERTS, NUM_EXPERTS), 1)
    ut = (ut_r < ut_c).astype(jnp.float32)
    bstart8 = jnp.dot(blocks8, ut, preferred_element_type=jnp.float32)
    base = bstart8[0:1, :] * BLK

    posmat = base + csum - 1.0
    efirst = jnp.minimum(i1, i2)
    elast = jnp.maximum(i1, i2)
    pos0 = jnp.sum(jnp.where(lane == efirst, posmat, 0.0), axis=1, keepdims=True)
    pos1 = jnp.sum(jnp.where(lane == elast, posmat, 0.0), axis=1, keepdims=True)
    w0 = jnp.sum(jnp.where(lane == efirst, w, 0.0), axis=1, keepdims=True)
    w1 = jnp.sum(jnp.where(lane == elast, w, 0.0), axis=1, keepdims=True)
    lane8 = lax.broadcasted_iota(jnp.int32, (TOKENS, 8), 1)
    pospair_ref[...] = jnp.where(
        lane8 == 0, pos0.astype(jnp.int32),
        jnp.where(lane8 == 1, pos1.astype(jnp.int32), 0))
    w0m_ref[...] = jnp.broadcast_to(w0, (TOKENS, NUM_EXPERTS))
    w1m_ref[...] = jnp.broadcast_to(w1, (TOKENS, NUM_EXPERTS))
    counts_ref[...] = jnp.broadcast_to(counts_i, (8, NUM_EXPERTS))


def _router_meta(x, gate_w):
    return pl.pallas_call(
        _router_meta_body,
        out_shape=(
            jax.ShapeDtypeStruct((TOKENS, 8), jnp.int32),
            jax.ShapeDtypeStruct((TOKENS, NUM_EXPERTS), jnp.float32),
            jax.ShapeDtypeStruct((TOKENS, NUM_EXPERTS), jnp.float32),
            jax.ShapeDtypeStruct((8, NUM_EXPERTS), jnp.int32),
        ),
    )(x, gate_w)


def _route_glue(pospair, counts8):
    counts = counts8[0]
    blocks = (counts + BLK - 1) // BLK
    blockstart = jnp.concatenate([jnp.zeros((1,), jnp.int32),
                                  jnp.cumsum(blocks)[:-1]]).astype(jnp.int32)
    bidx = jnp.arange(NB, dtype=jnp.int32)
    block_expert = jnp.sum(
        (blockstart[None, :] <= bidx[:, None]).astype(jnp.int32), axis=1) - 1
    block_expert = jnp.clip(block_expert, 0, NUM_EXPERTS - 1)
    block_valid = jnp.clip(
        counts[block_expert] - (bidx - blockstart[block_expert]) * BLK, 0, BLK)

    pos0 = pospair[:, 0]
    pos1 = pospair[:, 1]
    return block_expert, block_valid, pos0, pos1


def _sc_mesh():
    return plsc.VectorSubcoreMesh(core_axis_name="c", subcore_axis_name="s")


_DISP_TOK = TOKENS // _NW


@functools.cache
def _make_sc_dispatch():
    @functools.partial(
        pl.kernel,
        mesh=_sc_mesh(),
        out_type=jax.ShapeDtypeStruct((NR, HIDDEN), jnp.float32),
        scratch_types=[
            pltpu.VMEM((_DISP_TOK,), jnp.int32),
            pltpu.VMEM((_DISP_TOK,), jnp.int32),
            pltpu.VMEM((_DISP_TOK, HIDDEN), jnp.float32),
            pltpu.SemaphoreType.DMA,
            pltpu.SemaphoreType.DMA,
        ],
    )
    def _sc_dispatch(hid_hbm, pos0_hbm, pos1_hbm, out_hbm,
                     idx0_v, idx1_v, rows_v, sem0, sem1):
        wid = lax.axis_index("s") * _SC_CORES + lax.axis_index("c")
        tok_base = wid * _DISP_TOK
        pltpu.sync_copy(pos0_hbm.at[wid], idx0_v)
        pltpu.sync_copy(pos1_hbm.at[wid], idx1_v)
        pltpu.sync_copy(hid_hbm.at[pl.ds(tok_base, _DISP_TOK)], rows_v)
        cp0 = pltpu.async_copy(rows_v, out_hbm.at[idx0_v], sem0)
        cp1 = pltpu.async_copy(rows_v, out_hbm.at[idx1_v], sem1)
        cp0.wait()
        cp1.wait()

    return _sc_dispatch


def _group_ffn_body(be_ref, valid_ref, x_ref, wg_ref, wu_ref, wd_ref, out_ref):
    b = pl.program_id(0)

    @pl.when(valid_ref[b] > 0)
    def _():
        x = x_ref[...]
        g = jnp.dot(x, wg_ref[0], preferred_element_type=jnp.float32)
        u = jnp.dot(x, wu_ref[0], preferred_element_type=jnp.float32)
        h = (g * jax.nn.sigmoid(g)) * u
        out_ref[...] = jnp.dot(h, wd_ref[0], preferred_element_type=jnp.float32)


def _group_ffn(x_pad, w_gate, w_up, w_down, block_expert, block_valid):
    grid_spec = pltpu.PrefetchScalarGridSpec(
        num_scalar_prefetch=2,
        grid=(NB,),
        in_specs=[
            pl.BlockSpec((BLK, HIDDEN),
                         lambda b, be, vd: (jnp.where(vd[b] > 0, b, NB - 1), 0)),
            pl.BlockSpec((1, HIDDEN, MOE_FF), lambda b, be, vd: (be[b], 0, 0)),
            pl.BlockSpec((1, HIDDEN, MOE_FF), lambda b, be, vd: (be[b], 0, 0)),
            pl.BlockSpec((1, MOE_FF, HIDDEN), lambda b, be, vd: (be[b], 0, 0)),
        ],
        out_specs=pl.BlockSpec(
            (BLK, HIDDEN), lambda b, be, vd: (jnp.where(vd[b] > 0, b, NB - 1), 0)),
    )
    return pl.pallas_call(
        _group_ffn_body,
        grid_spec=grid_spec,
        out_shape=jax.ShapeDtypeStruct((NR, HIDDEN), jnp.float32),
    )(block_expert, block_valid, x_pad, w_gate, w_up, w_down)


_COMBINE_TOKENS = TOKENS // _NW


@functools.cache
def _make_sc_combine():
    @functools.partial(
        pl.kernel,
        mesh=_sc_mesh(),
        out_type=jax.ShapeDtypeStruct((TOKENS, HIDDEN), jnp.float32),
        scratch_types=[
            pltpu.VMEM((_COMBINE_TOKENS,), jnp.int32),
            pltpu.VMEM((_COMBINE_TOKENS,), jnp.int32),
            pltpu.VMEM((_COMBINE_TOKENS, NUM_EXPERTS), jnp.float32),
            pltpu.VMEM((_COMBINE_TOKENS, NUM_EXPERTS), jnp.float32),
            pltpu.VMEM((_COMBINE_CHUNK, HIDDEN), jnp.float32),
            pltpu.VMEM((_COMBINE_CHUNK, HIDDEN), jnp.float32),
            pltpu.VMEM((_COMBINE_CHUNK, HIDDEN), jnp.float32),
            pltpu.SemaphoreType.DMA,
            pltpu.SemaphoreType.DMA,
        ],
    )
    def _sc_combine(y_hbm, pos0_hbm, pos1_hbm, w0_hbm, w1_hbm, out_hbm,
                    pos0_v, pos1_v, w0_v, w1_v, y0_v, y1_v, o_v, sem0, sem1):
        wid = lax.axis_index("s") * _SC_CORES + lax.axis_index("c")
        tok_base = wid * _COMBINE_TOKENS
        pltpu.sync_copy(pos0_hbm.at[pl.ds(tok_base, _COMBINE_TOKENS)], pos0_v)
        pltpu.sync_copy(pos1_hbm.at[pl.ds(tok_base, _COMBINE_TOKENS)], pos1_v)
        pltpu.sync_copy(w0_hbm.at[pl.ds(tok_base, _COMBINE_TOKENS)], w0_v)
        pltpu.sync_copy(w1_hbm.at[pl.ds(tok_base, _COMBINE_TOKENS)], w1_v)

        def chunk(c, _):
            off = c * _COMBINE_CHUNK
            cp0 = pltpu.async_copy(
                y_hbm.at[pos0_v.at[pl.ds(off, _COMBINE_CHUNK)]], y0_v, sem0)
            cp1 = pltpu.async_copy(
                y_hbm.at[pos1_v.at[pl.ds(off, _COMBINE_CHUNK)]], y1_v, sem1)
            cp0.wait()
            cp1.wait()

            def token(j, _):
                wa = w0_v[off + j, :]
                wb = w1_v[off + j, :]
                for i in range(HIDDEN // 16):
                    sl = pl.ds(i * 16, 16)
                    o_v[j, sl] = wa * y0_v[j, sl] + wb * y1_v[j, sl]
                return 0

            lax.fori_loop(0, _COMBINE_CHUNK, token, 0)
            pltpu.sync_copy(
                o_v, out_hbm.at[pl.ds(tok_base + off, _COMBINE_CHUNK)])
            return 0

        lax.fori_loop(0, _COMBINE_TOKENS // _COMBINE_CHUNK, chunk, 0)

    return _sc_combine


@jax.jit
def kernel(hidden_states, gate_w, w_gate, w_up, w_down):
    pospair, w0m, w1m, counts8 = _router_meta(hidden_states, gate_w)
    block_expert, block_valid, pos0, pos1 = _route_glue(pospair, counts8)
    x_pad = _make_sc_dispatch()(
        hidden_states, pos0.reshape(_NW, _DISP_TOK), pos1.reshape(_NW, _DISP_TOK))
    y_pad = _group_ffn(x_pad, w_gate, w_up, w_down, block_expert, block_valid)
    return _make_sc_combine()(y_pad, pos0, pos1, w0m, w1m)


# speedup vs baseline: 3.0490x; 1.1231x over previous
import functools

import jax
import jax.numpy as jnp
from jax import lax
from jax.experimental import pallas as pl
from jax.experimental.pallas import tpu as pltpu
from jax.experimental.pallas import tpu_sc as plsc

NUM_EXPERTS = 16
TOP_K = 2
HIDDEN = 1024
MOE_FF = 768
TOKENS = 2048

NUM_ASSIGN = TOKENS * TOP_K
BLK = 256
NB = NUM_ASSIGN // BLK + NUM_EXPERTS
NR = NB * BLK

_SC_CORES = 2
_SC_SUBCORES = 16
_NW = _SC_CORES * _SC_SUBCORES

_DISPATCH_ROWS = NR // _NW
_DISPATCH_CHUNK = 48
_COMBINE_CHUNK = 32


_CS_CHUNK = 256


def _router_meta_body(x_ref, gate_ref, pospair_ref,
                      w0m_ref, w1m_ref, counts_ref):
    logits = jnp.dot(x_ref[...], gate_ref[...], preferred_element_type=jnp.float32)
    probs = jax.nn.softmax(logits, axis=-1)
    lane = lax.broadcasted_iota(jnp.int32, probs.shape, 1)
    m1 = jnp.max(probs, axis=-1, keepdims=True)
    i1 = jnp.min(jnp.where(probs == m1, lane, NUM_EXPERTS), axis=-1, keepdims=True)
    masked = jnp.where(lane == i1, -jnp.inf, probs)
    m2 = jnp.max(masked, axis=-1, keepdims=True)
    i2 = jnp.min(jnp.where(masked == m2, lane, NUM_EXPERTS), axis=-1, keepdims=True)
    denom = m1 + m2
    sel = (lane == i1) | (lane == i2)
    w = jnp.where(lane == i1, m1, jnp.where(lane == i2, m2, 0.0)) / denom

    m = sel.astype(jnp.float32)
    tri_r = lax.broadcasted_iota(jnp.int32, (_CS_CHUNK, _CS_CHUNK), 0)
    tri_c = lax.broadcasted_iota(jnp.int32, (_CS_CHUNK, _CS_CHUNK), 1)
    tri = (tri_r >= tri_c).astype(jnp.float32)
    carry = jnp.zeros((1, NUM_EXPERTS), jnp.float32)
    chunks = []
    for c in range(TOKENS // _CS_CHUNK):
        mc = m[c * _CS_CHUNK:(c + 1) * _CS_CHUNK]
        cs = jnp.dot(tri, mc, preferred_element_type=jnp.float32) + carry
        chunks.append(cs)
        carry = cs[_CS_CHUNK - 1:_CS_CHUNK, :]
    csum = jnp.concatenate(chunks, axis=0)
    counts = carry

    counts_i = counts.astype(jnp.int32)
    blocks_f = ((counts_i + (BLK - 1)) >> (BLK.bit_length() - 1)).astype(
        jnp.float32)
    blocks8 = jnp.broadcast_to(blocks_f, (8, NUM_EXPERTS))
    ut_r = lax.broadcasted_iota(jnp.int32, (NUM_EXPERTS, NUM_EXPERTS), 0)
    ut_c = lax.broadcasted_iota(jnp.int32, (NUM_EXPERTS, NUM_EXPERTS), 1)
    ut = (ut_r < ut_c).astype(jnp.float32)
    bstart8 = jnp.dot(blocks8, ut, preferred_element_type=jnp.float32)
    base = bstart8[0:1, :] * BLK

    posmat = base + csum - 1.0
    efirst = jnp.minimum(i1, i2)
    elast = jnp.maximum(i1, i2)
    pos0 = jnp.sum(jnp.where(lane == efirst, posmat, 0.0), axis=1, keepdims=True)
    pos1 = jnp.sum(jnp.where(lane == elast, posmat, 0.0), axis=1, keepdims=True)
    w0 = jnp.sum(jnp.where(lane == efirst, w, 0.0), axis=1, keepdims=True)
    w1 = jnp.sum(jnp.where(lane == elast, w, 0.0), axis=1, keepdims=True)
    lane8 = lax.broadcasted_iota(jnp.int32, (TOKENS, 8), 1)
    pospair_ref[...] = jnp.where(
        lane8 == 0, pos0.astype(jnp.int32),
        jnp.where(lane8 == 1, pos1.astype(jnp.int32), 0))
    w0m_ref[...] = jnp.broadcast_to(w0, (TOKENS, NUM_EXPERTS))
    w1m_ref[...] = jnp.broadcast_to(w1, (TOKENS, NUM_EXPERTS))
    counts_ref[...] = jnp.broadcast_to(counts_i, (8, NUM_EXPERTS))


def _router_meta(x, gate_w):
    return pl.pallas_call(
        _router_meta_body,
        out_shape=(
            jax.ShapeDtypeStruct((TOKENS, 8), jnp.int32),
            jax.ShapeDtypeStruct((TOKENS, NUM_EXPERTS), jnp.float32),
            jax.ShapeDtypeStruct((TOKENS, NUM_EXPERTS), jnp.float32),
            jax.ShapeDtypeStruct((8, NUM_EXPERTS), jnp.int32),
        ),
    )(x, gate_w)


def _route_glue(pospair, counts8):
    counts = counts8[0]
    blocks = (counts + BLK - 1) // BLK
    blockstart = jnp.concatenate([jnp.zeros((1,), jnp.int32),
                                  jnp.cumsum(blocks)[:-1]]).astype(jnp.int32)
    bidx = jnp.arange(NB, dtype=jnp.int32)
    block_expert = jnp.sum(
        (blockstart[None, :] <= bidx[:, None]).astype(jnp.int32), axis=1) - 1
    block_expert = jnp.clip(block_expert, 0, NUM_EXPERTS - 1)
    block_valid = jnp.clip(
        counts[block_expert] - (bidx - blockstart[block_expert]) * BLK, 0, BLK)

    pos0 = pospair[:, 0]
    pos1 = pospair[:, 1]
    return block_expert, block_valid, pos0, pos1


def _sc_mesh():
    return plsc.VectorSubcoreMesh(core_axis_name="c", subcore_axis_name="s")


_DISP_TOK = TOKENS // _NW


@functools.cache
def _make_sc_dispatch():
    @functools.partial(
        pl.kernel,
        mesh=_sc_mesh(),
        out_type=jax.ShapeDtypeStruct((NR, HIDDEN), jnp.float32),
        scratch_types=[
            pltpu.VMEM((_DISP_TOK,), jnp.int32),
            pltpu.VMEM((_DISP_TOK,), jnp.int32),
            pltpu.VMEM((_DISP_TOK, HIDDEN), jnp.float32),
            pltpu.SemaphoreType.DMA,
            pltpu.SemaphoreType.DMA,
        ],
    )
    def _sc_dispatch(hid_hbm, pos0_hbm, pos1_hbm, out_hbm,
                     idx0_v, idx1_v, rows_v, sem0, sem1):
        wid = lax.axis_index("s") * _SC_CORES + lax.axis_index("c")
        tok_base = wid * _DISP_TOK
        pltpu.sync_copy(pos0_hbm.at[wid], idx0_v)
        pltpu.sync_copy(pos1_hbm.at[wid], idx1_v)
        pltpu.sync_copy(hid_hbm.at[pl.ds(tok_base, _DISP_TOK)], rows_v)
        cp0 = pltpu.async_copy(rows_v, out_hbm.at[idx0_v], sem0)
        cp1 = pltpu.async_copy(rows_v, out_hbm.at[idx1_v], sem1)
        cp0.wait()
        cp1.wait()

    return _sc_dispatch


def _group_ffn_body(be_ref, valid_ref, x_ref, wg_ref, wu_ref, wd_ref, out_ref):
    b = pl.program_id(0)

    @pl.when(valid_ref[b] > 0)
    def _():
        x = x_ref[...]
        g = jnp.dot(x, wg_ref[0], preferred_element_type=jnp.float32)
        u = jnp.dot(x, wu_ref[0], preferred_element_type=jnp.float32)
        h = (g * jax.nn.sigmoid(g)) * u
        out_ref[...] = jnp.dot(h, wd_ref[0], preferred_element_type=jnp.float32)


def _group_ffn(x_pad, w_gate, w_up, w_down, block_expert, block_valid):
    grid_spec = pltpu.PrefetchScalarGridSpec(
        num_scalar_prefetch=2,
        grid=(NB,),
        in_specs=[
            pl.BlockSpec((BLK, HIDDEN),
                         lambda b, be, vd: (jnp.where(vd[b] > 0, b, NB - 1), 0)),
            pl.BlockSpec((1, HIDDEN, MOE_FF), lambda b, be, vd: (be[b], 0, 0)),
            pl.BlockSpec((1, HIDDEN, MOE_FF), lambda b, be, vd: (be[b], 0, 0)),
            pl.BlockSpec((1, MOE_FF, HIDDEN), lambda b, be, vd: (be[b], 0, 0)),
        ],
        out_specs=pl.BlockSpec(
            (BLK, HIDDEN), lambda b, be, vd: (jnp.where(vd[b] > 0, b, NB - 1), 0)),
    )
    return pl.pallas_call(
        _group_ffn_body,
        grid_spec=grid_spec,
        out_shape=jax.ShapeDtypeStruct((NR, HIDDEN), jnp.float32),
    )(block_expert, block_valid, x_pad, w_gate, w_up, w_down)


_COMBINE_TOKENS = TOKENS // _NW


@functools.cache
def _make_sc_combine():
    @functools.partial(
        pl.kernel,
        mesh=_sc_mesh(),
        out_type=jax.ShapeDtypeStruct((TOKENS, HIDDEN), jnp.float32),
        scratch_types=[
            pltpu.VMEM((_COMBINE_TOKENS,), jnp.int32),
            pltpu.VMEM((_COMBINE_TOKENS,), jnp.int32),
            pltpu.VMEM((_COMBINE_TOKENS, NUM_EXPERTS), jnp.float32),
            pltpu.VMEM((_COMBINE_TOKENS, NUM_EXPERTS), jnp.float32),
            pltpu.VMEM((_COMBINE_CHUNK, HIDDEN), jnp.float32),
            pltpu.VMEM((_COMBINE_CHUNK, HIDDEN), jnp.float32),
            pltpu.VMEM((_COMBINE_CHUNK, HIDDEN), jnp.float32),
            pltpu.SemaphoreType.DMA,
            pltpu.SemaphoreType.DMA,
        ],
    )
    def _sc_combine(y_hbm, pos0_hbm, pos1_hbm, w0_hbm, w1_hbm, out_hbm,
                    pos0_v, pos1_v, w0_v, w1_v, y0_v, y1_v, o_v, sem0, sem1):
        wid = lax.axis_index("s") * _SC_CORES + lax.axis_index("c")
        tok_base = wid * _COMBINE_TOKENS
        pltpu.sync_copy(pos0_hbm.at[pl.ds(tok_base, _COMBINE_TOKENS)], pos0_v)
        pltpu.sync_copy(pos1_hbm.at[pl.ds(tok_base, _COMBINE_TOKENS)], pos1_v)
        pltpu.sync_copy(w0_hbm.at[pl.ds(tok_base, _COMBINE_TOKENS)], w0_v)
        pltpu.sync_copy(w1_hbm.at[pl.ds(tok_base, _COMBINE_TOKENS)], w1_v)

        def chunk(c, _):
            off = c * _COMBINE_CHUNK
            cp0 = pltpu.async_copy(
                y_hbm.at[pos0_v.at[pl.ds(off, _COMBINE_CHUNK)]], y0_v, sem0)
            cp1 = pltpu.async_copy(
                y_hbm.at[pos1_v.at[pl.ds(off, _COMBINE_CHUNK)]], y1_v, sem1)
            cp0.wait()
            cp1.wait()

            def token(j, _):
                wa = w0_v[off + j, :]
                wb = w1_v[off + j, :]
                for i in range(HIDDEN // 16):
                    sl = pl.ds(i * 16, 16)
                    o_v[j, sl] = wa * y0_v[j, sl] + wb * y1_v[j, sl]
                return 0

            lax.fori_loop(0, _COMBINE_CHUNK, token, 0)
            pltpu.sync_copy(
                o_v, out_hbm.at[pl.ds(tok_base + off, _COMBINE_CHUNK)])
            return 0

        lax.fori_loop(0, _COMBINE_TOKENS // _COMBINE_CHUNK, chunk, 0)

    return _sc_combine


@jax.jit
def kernel(hidden_states, gate_w, w_gate, w_up, w_down):
    pospair, w0m, w1m, counts8 = _router_meta(hidden_states, gate_w)
    block_expert, block_valid, pos0, pos1 = _route_glue(pospair, counts8)
    x_pad = _make_sc_dispatch()(
        hidden_states, pos0.reshape(_NW, _DISP_TOK), pos1.reshape(_NW, _DISP_TOK))
    y_pad = _group_ffn(x_pad, w_gate, w_up, w_down, block_expert, block_valid)
    return _make_sc_combine()(y_pad, pos0, pos1, w0m, w1m)
